# Initial kernel scaffold; baseline (speedup 1.0000x reference)
import jax
import jax.numpy as jnp
from jax.experimental import pallas as pl


def kernel(node_feat, edge_index, edge_type, g_idx, w_n2l_W, w_n2l_b, conv_W, conv_b, merge_W, merge_b, ro_W, ro_b):
    raise NotImplementedError("write your pallas kernel here")


# speedup vs baseline: 15.3013x; 15.3013x over previous
import functools

import jax
import jax.numpy as jnp
from jax import lax
from jax.experimental import pallas as pl
from jax.experimental.pallas import tpu as pltpu
from jax.experimental.pallas import tpu_sc as plsc

N = 10000
E = 320000
D = 128
T = 4
LV = 3
G = 64

RB = 1000
NBLK = N // RB
Q = 4
DQ = D // Q
NT = N * T
ROWS_PER_SUB = NT // 16
BATCH = 125
SUBBATCH = 8
EDGE_ROWS = E // BATCH
ROWS_PER_WORKER = EDGE_ROWS // 16
NBATCH = ROWS_PER_WORKER // SUBBATCH


def _lin_tanh_body(x_ref, w_ref, b_ref, o_ref):
    o_ref[...] = jnp.tanh(
        jnp.dot(x_ref[...], w_ref[...], preferred_element_type=jnp.float32)
        + b_ref[...])


def _lin_tanh(x, w, b):
    k_in, k_out = w.shape
    return pl.pallas_call(
        _lin_tanh_body,
        grid=(NBLK,),
        in_specs=[
            pl.BlockSpec((RB, k_in), lambda i: (i, 0)),
            pl.BlockSpec((k_in, k_out), lambda i: (0, 0)),
            pl.BlockSpec((1, k_out), lambda i: (0, 0)),
        ],
        out_specs=pl.BlockSpec((RB, k_out), lambda i: (i, 0)),
        out_shape=jax.ShapeDtypeStruct((N, k_out), jnp.float32),
    )(x, w, b.reshape(1, k_out))


def _conv_body(x_ref, w_ref, b_ref, o_ref):
    o_ref[...] = (
        jnp.dot(x_ref[...], w_ref[...], preferred_element_type=jnp.float32)
        + b_ref[...])


def _conv(h, w, b):
    return pl.pallas_call(
        _conv_body,
        grid=(NBLK,),
        in_specs=[
            pl.BlockSpec((RB, D), lambda i: (i, 0)),
            pl.BlockSpec((D, T * D), lambda i: (0, 0)),
            pl.BlockSpec((1, T * D), lambda i: (0, 0)),
        ],
        out_specs=pl.BlockSpec((RB, T * D), lambda i: (i, 0)),
        out_shape=jax.ShapeDtypeStruct((N, T * D), jnp.float32),
    )(h, w, b.reshape(1, T * D))


def _merge_body(a0_ref, a1_ref, a2_ref, a3_ref, h_ref, w_ref, b_ref, o_ref):
    cur = b_ref[...] + h_ref[...]
    for qq, a_ref in enumerate((a0_ref, a1_ref, a2_ref, a3_ref)):
        cur = cur + jnp.dot(jnp.tanh(a_ref[...]),
                            w_ref[pl.ds(qq * D, D), :],
                            preferred_element_type=jnp.float32)
    o_ref[...] = jnp.tanh(cur)


def _merge(aggs, h, w_perm, b):
    return pl.pallas_call(
        _merge_body,
        grid=(NBLK,),
        in_specs=[
            pl.BlockSpec((RB, D), lambda i: (i, 0)),
            pl.BlockSpec((RB, D), lambda i: (i, 0)),
            pl.BlockSpec((RB, D), lambda i: (i, 0)),
            pl.BlockSpec((RB, D), lambda i: (i, 0)),
            pl.BlockSpec((RB, D), lambda i: (i, 0)),
            pl.BlockSpec((T * D, D), lambda i: (0, 0)),
            pl.BlockSpec((1, D), lambda i: (0, 0)),
        ],
        out_specs=pl.BlockSpec((RB, D), lambda i: (i, 0)),
        out_shape=jax.ShapeDtypeStruct((N, D), jnp.float32),
    )(*aggs, h, w_perm, b.reshape(1, D))


def _readout_body(gb_ref, h_ref, g_ref, w_ref, b_ref, o_ref, acc_ref):
    i = pl.program_id(0)

    @pl.when(i == 0)
    def _():
        acc_ref[...] = jnp.full((G, D), -jnp.inf, jnp.float32)

    g = g_ref[...]
    h = h_ref[...]
    lo = gb_ref[2 * i]
    hi = gb_ref[2 * i + 1]
    rowids = lax.broadcasted_iota(jnp.int32, (G, 1), 0)

    def body(gg, carry):
        m = jnp.max(jnp.where(g == gg, h, -jnp.inf), axis=0, keepdims=True)
        acc_ref[...] = jnp.where(rowids == gg,
                                 jnp.maximum(acc_ref[...], m), acc_ref[...])
        return carry

    lax.fori_loop(lo, hi + 1, body, 0)

    @pl.when(i == NBLK - 1)
    def _():
        ge = acc_ref[...]
        ge = jnp.where(jnp.isfinite(ge), ge, 0.0)
        o_ref[...] = jnp.tanh(
            jnp.dot(ge, w_ref[...], preferred_element_type=jnp.float32)
            + b_ref[...])


def _readout(h, g_idx, w, b):
    g2 = g_idx.reshape(NBLK, RB)
    gb = jnp.stack([g2[:, 0], g2[:, RB - 1]], axis=1).reshape(2 * NBLK)
    grid_spec = pltpu.PrefetchScalarGridSpec(
        num_scalar_prefetch=1,
        grid=(NBLK,),
        in_specs=[
            pl.BlockSpec((RB, D), lambda i, gb: (i, 0)),
            pl.BlockSpec((RB, 1), lambda i, gb: (i, 0)),
            pl.BlockSpec((D, D), lambda i, gb: (0, 0)),
            pl.BlockSpec((1, D), lambda i, gb: (0, 0)),
        ],
        out_specs=pl.BlockSpec((G, D), lambda i, gb: (0, 0)),
        scratch_shapes=[pltpu.VMEM((G, D), jnp.float32)],
    )
    return pl.pallas_call(
        _readout_body,
        grid_spec=grid_spec,
        out_shape=jax.ShapeDtypeStruct((G, D), jnp.float32),
        compiler_params=pltpu.CompilerParams(
            dimension_semantics=("arbitrary",)),
    )(gb, h, g_idx.reshape(N, 1), w, b.reshape(1, D))


_SC_MESH = plsc.VectorSubcoreMesh(core_axis_name="c", subcore_axis_name="s")


@functools.partial(
    pl.kernel,
    out_type=jax.ShapeDtypeStruct((Q, NT, DQ), jnp.float32),
    mesh=_SC_MESH,
    compiler_params=pltpu.CompilerParams(use_tc_tiling_on_sc=False),
    scratch_types=[
        pltpu.VMEM((SUBBATCH, BATCH), jnp.int32),
        pltpu.VMEM((SUBBATCH, BATCH), jnp.int32),
        pltpu.VMEM((SUBBATCH, BATCH, DQ), jnp.float32),
        pltpu.VMEM_SHARED((NT, DQ), jnp.float32),
        pltpu.SemaphoreType.DMA((SUBBATCH,)),
        pltpu.SemaphoreType.DMA,
    ],
)
def _sc_spmm(table_hbm, srcg_hbm, dstf_hbm, zeros_hbm, agg_hbm,
             sidx_v, didx_v, rows_v, acc_sh, gsem, ssem):
    c = lax.axis_index("c")
    s = lax.axis_index("s")
    for qi in range(2):
        q = c * 2 + qi
        pltpu.sync_copy(zeros_hbm, acc_sh.at[pl.ds(s * ROWS_PER_SUB,
                                                   ROWS_PER_SUB), :])
        plsc.subcore_barrier()

        def batch_body(b, carry):
            row0 = s * ROWS_PER_WORKER + b * SUBBATCH
            pltpu.sync_copy(srcg_hbm.at[q, pl.ds(row0, SUBBATCH)], sidx_v)
            pltpu.sync_copy(dstf_hbm.at[pl.ds(row0, SUBBATCH)], didx_v)
            gds = [pltpu.async_copy(table_hbm.at[sidx_v.at[j]],
                                    rows_v.at[j], gsem.at[j])
                   for j in range(SUBBATCH)]
            sds = []
            for j in range(SUBBATCH):
                gds[j].wait()
                sds.append(pltpu.async_copy(rows_v.at[j],
                                            acc_sh.at[didx_v.at[j]],
                                            ssem, add=True))
            for sd in sds:
                sd.wait()
            return carry

        lax.fori_loop(0, NBATCH, batch_body, 0)
        plsc.subcore_barrier()
        pltpu.sync_copy(
            acc_sh.at[pl.ds(s * ROWS_PER_SUB, ROWS_PER_SUB), :],
            agg_hbm.at[q, pl.ds(s * ROWS_PER_SUB, ROWS_PER_SUB), :])
        plsc.subcore_barrier()


def kernel(node_feat, edge_index, edge_type, g_idx,
           w_n2l_W, w_n2l_b, conv_W, conv_b, merge_W, merge_b, ro_W, ro_b):
    src = edge_index[0]
    dst = edge_index[1]
    base = src * T + edge_type
    srcg = (base[None, :] * Q
            + jnp.arange(Q, dtype=jnp.int32)[:, None])
    srcg = srcg.reshape(Q, EDGE_ROWS, BATCH)
    dstf = (dst * T + edge_type).reshape(EDGE_ROWS, BATCH)
    zeros_blk = jnp.zeros((ROWS_PER_SUB, DQ), jnp.float32)
    mw_perm = (merge_W.reshape(LV, T, Q, DQ, D)
               .transpose(0, 2, 1, 3, 4).reshape(LV, T * D, D))

    h = _lin_tanh(node_feat, w_n2l_W, w_n2l_b)
    for lv in range(LV):
        conv = _conv(h, conv_W[lv], conv_b[lv])
        table = conv.reshape(N * T * Q, DQ)
        agg4 = _sc_spmm(table, srcg, dstf, zeros_blk)
        aggs = [agg4[qq].reshape(N, D) for qq in range(Q)]
        h = _merge(aggs, h, mw_perm[lv], merge_b[lv])
    return _readout(h, g_idx, ro_W, ro_b)


# speedup vs baseline: 17.0466x; 1.1141x over previous
import functools

import jax
import jax.numpy as jnp
from jax import lax
from jax.experimental import pallas as pl
from jax.experimental.pallas import tpu as pltpu
from jax.experimental.pallas import tpu_sc as plsc

N = 10000
E = 320000
D = 128
T = 4
LV = 3
G = 64

RB = 1000
NBLK = N // RB
Q = 4
DQ = D // Q
NT = N * T
ROWS_PER_SUB = NT // 16
BATCH = 125
SUBBATCH = 10
EDGE_ROWS = E // BATCH
ROWS_PER_WORKER = EDGE_ROWS // 16
NBATCH = ROWS_PER_WORKER // SUBBATCH


def _lin_tanh_body(x_ref, w_ref, b_ref, o_ref):
    o_ref[...] = jnp.tanh(
        jnp.dot(x_ref[...], w_ref[...], preferred_element_type=jnp.float32)
        + b_ref[...])


def _lin_tanh(x, w, b):
    k_in, k_out = w.shape
    return pl.pallas_call(
        _lin_tanh_body,
        grid=(NBLK,),
        in_specs=[
            pl.BlockSpec((RB, k_in), lambda i: (i, 0)),
            pl.BlockSpec((k_in, k_out), lambda i: (0, 0)),
            pl.BlockSpec((1, k_out), lambda i: (0, 0)),
        ],
        out_specs=pl.BlockSpec((RB, k_out), lambda i: (i, 0)),
        out_shape=jax.ShapeDtypeStruct((N, k_out), jnp.float32),
    )(x, w, b.reshape(1, k_out))


def _conv_body(x_ref, w_ref, b_ref, o_ref):
    x = x_ref[...]
    for qq in range(Q):
        o_ref[qq] = (
            jnp.dot(x, w_ref[qq], preferred_element_type=jnp.float32)
            + b_ref[qq])


def _conv(h, w_perm, b_perm):
    return pl.pallas_call(
        _conv_body,
        grid=(NBLK,),
        in_specs=[
            pl.BlockSpec((RB, D), lambda i: (i, 0)),
            pl.BlockSpec((Q, D, D), lambda i: (0, 0, 0)),
            pl.BlockSpec((Q, 1, D), lambda i: (0, 0, 0)),
        ],
        out_specs=pl.BlockSpec((Q, RB, D), lambda i: (0, i, 0)),
        out_shape=jax.ShapeDtypeStruct((Q, N, D), jnp.float32),
    )(h, w_perm, b_perm)


def _merge_body(a0_ref, a1_ref, a2_ref, a3_ref, h_ref, w_ref, b_ref, o_ref):
    cur = b_ref[...] + h_ref[...]
    for qq, a_ref in enumerate((a0_ref, a1_ref, a2_ref, a3_ref)):
        cur = cur + jnp.dot(jnp.tanh(a_ref[...]),
                            w_ref[pl.ds(qq * D, D), :],
                            preferred_element_type=jnp.float32)
    o_ref[...] = jnp.tanh(cur)


def _merge(aggs, h, w_perm, b):
    return pl.pallas_call(
        _merge_body,
        grid=(NBLK,),
        in_specs=[
            pl.BlockSpec((RB, D), lambda i: (i, 0)),
            pl.BlockSpec((RB, D), lambda i: (i, 0)),
            pl.BlockSpec((RB, D), lambda i: (i, 0)),
            pl.BlockSpec((RB, D), lambda i: (i, 0)),
            pl.BlockSpec((RB, D), lambda i: (i, 0)),
            pl.BlockSpec((T * D, D), lambda i: (0, 0)),
            pl.BlockSpec((1, D), lambda i: (0, 0)),
        ],
        out_specs=pl.BlockSpec((RB, D), lambda i: (i, 0)),
        out_shape=jax.ShapeDtypeStruct((N, D), jnp.float32),
    )(*aggs, h, w_perm, b.reshape(1, D))


def _readout_body(gb_ref, h_ref, g_ref, w_ref, b_ref, o_ref, acc_ref):
    i = pl.program_id(0)

    @pl.when(i == 0)
    def _():
        acc_ref[...] = jnp.full((G, D), -jnp.inf, jnp.float32)

    g = g_ref[...]
    h = h_ref[...]
    lo = gb_ref[2 * i]
    hi = gb_ref[2 * i + 1]
    rowids = lax.broadcasted_iota(jnp.int32, (G, 1), 0)

    def body(gg, carry):
        m = jnp.max(jnp.where(g == gg, h, -jnp.inf), axis=0, keepdims=True)
        acc_ref[...] = jnp.where(rowids == gg,
                                 jnp.maximum(acc_ref[...], m), acc_ref[...])
        return carry

    lax.fori_loop(lo, hi + 1, body, 0)

    @pl.when(i == NBLK - 1)
    def _():
        ge = acc_ref[...]
        ge = jnp.where(jnp.isfinite(ge), ge, 0.0)
        o_ref[...] = jnp.tanh(
            jnp.dot(ge, w_ref[...], preferred_element_type=jnp.float32)
            + b_ref[...])


def _readout(h, g_idx, w, b):
    g2 = g_idx.reshape(NBLK, RB)
    gb = jnp.stack([g2[:, 0], g2[:, RB - 1]], axis=1).reshape(2 * NBLK)
    grid_spec = pltpu.PrefetchScalarGridSpec(
        num_scalar_prefetch=1,
        grid=(NBLK,),
        in_specs=[
            pl.BlockSpec((RB, D), lambda i, gb: (i, 0)),
            pl.BlockSpec((RB, 1), lambda i, gb: (i, 0)),
            pl.BlockSpec((D, D), lambda i, gb: (0, 0)),
            pl.BlockSpec((1, D), lambda i, gb: (0, 0)),
        ],
        out_specs=pl.BlockSpec((G, D), lambda i, gb: (0, 0)),
        scratch_shapes=[pltpu.VMEM((G, D), jnp.float32)],
    )
    return pl.pallas_call(
        _readout_body,
        grid_spec=grid_spec,
        out_shape=jax.ShapeDtypeStruct((G, D), jnp.float32),
        compiler_params=pltpu.CompilerParams(
            dimension_semantics=("arbitrary",)),
    )(gb, h, g_idx.reshape(N, 1), w, b.reshape(1, D))


_SC_MESH = plsc.VectorSubcoreMesh(core_axis_name="c", subcore_axis_name="s")


@functools.partial(
    pl.kernel,
    out_type=jax.ShapeDtypeStruct((Q, NT, DQ), jnp.float32),
    mesh=_SC_MESH,
    compiler_params=pltpu.CompilerParams(use_tc_tiling_on_sc=False),
    scratch_types=[
        pltpu.VMEM((SUBBATCH, BATCH), jnp.int32),
        pltpu.VMEM((SUBBATCH, BATCH), jnp.int32),
        pltpu.VMEM((SUBBATCH, BATCH, DQ), jnp.float32),
        pltpu.VMEM_SHARED((NT, DQ), jnp.float32),
        pltpu.SemaphoreType.DMA((SUBBATCH,)),
        pltpu.SemaphoreType.DMA,
    ],
)
def _sc_spmm(table_hbm, srcg_hbm, dstf_hbm, zeros_hbm, agg_hbm,
             sidx_v, didx_v, rows_v, acc_sh, gsem, ssem):
    c = lax.axis_index("c")
    s = lax.axis_index("s")
    for qi in range(2):
        q = c * 2 + qi
        pltpu.sync_copy(zeros_hbm, acc_sh.at[pl.ds(s * ROWS_PER_SUB,
                                                   ROWS_PER_SUB), :])
        plsc.subcore_barrier()

        def batch_body(b, carry):
            row0 = s * ROWS_PER_WORKER + b * SUBBATCH
            pltpu.sync_copy(srcg_hbm.at[q, pl.ds(row0, SUBBATCH)], sidx_v)
            pltpu.sync_copy(dstf_hbm.at[pl.ds(row0, SUBBATCH)], didx_v)
            gds = [pltpu.async_copy(table_hbm.at[sidx_v.at[j]],
                                    rows_v.at[j], gsem.at[j])
                   for j in range(SUBBATCH)]
            sds = []
            for j in range(SUBBATCH):
                gds[j].wait()
                sds.append(pltpu.async_copy(rows_v.at[j],
                                            acc_sh.at[didx_v.at[j]],
                                            ssem, add=True))
            for sd in sds:
                sd.wait()
            return carry

        lax.fori_loop(0, NBATCH, batch_body, 0)
        plsc.subcore_barrier()
        pltpu.sync_copy(
            acc_sh.at[pl.ds(s * ROWS_PER_SUB, ROWS_PER_SUB), :],
            agg_hbm.at[q, pl.ds(s * ROWS_PER_SUB, ROWS_PER_SUB), :])
        plsc.subcore_barrier()


def kernel(node_feat, edge_index, edge_type, g_idx,
           w_n2l_W, w_n2l_b, conv_W, conv_b, merge_W, merge_b, ro_W, ro_b):
    src = edge_index[0]
    dst = edge_index[1]
    base = src * T + edge_type
    srcg = (base[None, :]
            + (jnp.arange(Q, dtype=jnp.int32) * NT)[:, None])
    srcg = srcg.reshape(Q, EDGE_ROWS, BATCH)
    dstf = (dst * T + edge_type).reshape(EDGE_ROWS, BATCH)
    zeros_blk = jnp.zeros((ROWS_PER_SUB, DQ), jnp.float32)
    cw_perm = (conv_W.reshape(LV, D, T, Q, DQ)
               .transpose(0, 3, 1, 2, 4).reshape(LV, Q, D, D))
    cb_perm = (conv_b.reshape(LV, T, Q, DQ)
               .transpose(0, 2, 1, 3).reshape(LV, Q, 1, D))
    mw_perm = (merge_W.reshape(LV, T, Q, DQ, D)
               .transpose(0, 2, 1, 3, 4).reshape(LV, T * D, D))

    h = _lin_tanh(node_feat, w_n2l_W, w_n2l_b)
    for lv in range(LV):
        conv4 = _conv(h, cw_perm[lv], cb_perm[lv])
        table = conv4.reshape(N * T * Q, DQ)
        agg4 = _sc_spmm(table, srcg, dstf, zeros_blk)
        aggs = [agg4[qq].reshape(N, D) for qq in range(Q)]
        h = _merge(aggs, h, mw_perm[lv], merge_b[lv])
    return _readout(h, g_idx, ro_W, ro_b)
